# baseline (speedup 1.0000x reference)
import functools

import jax
import jax.numpy as jnp
from jax import lax
from jax.experimental import pallas as pl
from jax.experimental.pallas import tpu as pltpu
from jax.experimental.pallas import tpu_sc as plsc

NC = 2
NS = 16
NW = NC * NS

G = 16
NBUF = 2


@functools.cache
def _build(V, D, B0, B1):
  assert B0 % NW == 0
  rows_per_w = B0 // NW
  assert rows_per_w % (G * NBUF) == 0
  n_groups = rows_per_w // G
  mesh = plsc.VectorSubcoreMesh(
      core_axis_name="c", subcore_axis_name="s", num_cores=NC, num_subcores=NS)

  @functools.partial(
      pl.kernel,
      out_type=jax.ShapeDtypeStruct((B0, B1, D), jnp.float32),
      mesh=mesh,
      scratch_types=[
          pltpu.VMEM((rows_per_w, B1), jnp.int32),
          [pltpu.VMEM((G, B1, D), jnp.float32) for _ in range(NBUF)],
          [pltpu.SemaphoreType.DMA for _ in range(NBUF)],
          [pltpu.SemaphoreType.DMA for _ in range(NBUF)],
      ],
      compiler_params=pltpu.CompilerParams(use_tc_tiling_on_sc=False),
  )
  def gather_kernel(table_hbm, data_hbm, out_hbm, idx_v, rows, gsem, ssem):
    wid = lax.axis_index("s") * NC + lax.axis_index("c")
    row0 = wid * rows_per_w
    pltpu.sync_copy(data_hbm.at[pl.ds(row0, rows_per_w)], idx_v)

    def fire_group(g, b):
      for k in range(G):
        pltpu.async_copy(
            table_hbm.at[idx_v.at[g * G + k]], rows[b].at[k], gsem[b])

    def drain_group(g, b):
      pltpu.make_async_copy(
          out_hbm.at[pl.ds(row0 + g * G, G)], rows[b], gsem[b]).wait()

    def store_group(g, b):
      pltpu.async_copy(rows[b], out_hbm.at[pl.ds(row0 + g * G, G)], ssem[b])

    def wait_store(g, b):
      pltpu.make_async_copy(
          rows[b], out_hbm.at[pl.ds(row0 + g * G, G)], ssem[b]).wait()

    for b in range(NBUF):
      fire_group(b, b)

    def step(p, carry):
      for b in range(NBUF):
        g = p * NBUF + b
        drain_group(g, b)
        store_group(g, b)

        @pl.when(g + NBUF < n_groups)
        def _():
          wait_store(g, b)
          fire_group(g + NBUF, b)

      return carry

    lax.fori_loop(0, n_groups // NBUF, step, 0)
    for b in range(NBUF):
      wait_store(n_groups - NBUF + b, b)

  return gather_kernel


def kernel(data, ivectors):
  B0, B1 = data.shape
  V, D = ivectors.shape
  return _build(V, D, B0, B1)(ivectors, data.astype(jnp.int32))


# speedup vs baseline: 1.2362x; 1.2362x over previous
import functools

import jax
import jax.numpy as jnp
from jax import lax
from jax.experimental import pallas as pl
from jax.experimental.pallas import tpu as pltpu
from jax.experimental.pallas import tpu_sc as plsc

NC = 2
NS = 16
NW = NC * NS

G = 8
NBUF = 2
PADR = 56
PADD = 128


@functools.cache
def _build(V, D, B0, B1):
  assert B0 % NW == 0
  rows_per_w = B0 // NW
  assert rows_per_w % (G * NBUF) == 0
  n_groups = rows_per_w // G
  mesh = plsc.VectorSubcoreMesh(
      core_axis_name="c", subcore_axis_name="s", num_cores=NC, num_subcores=NS)

  @functools.partial(
      pl.kernel,
      out_type=jax.ShapeDtypeStruct((B0, PADR, PADD), jnp.float32),
      mesh=mesh,
      scratch_types=[
          pltpu.VMEM((rows_per_w, B1), jnp.int32),
          [pltpu.VMEM((G, B1, PADD), jnp.float32) for _ in range(NBUF)],
          [pltpu.SemaphoreType.DMA for _ in range(NBUF)],
          [pltpu.SemaphoreType.DMA for _ in range(NBUF)],
      ],
      compiler_params=pltpu.CompilerParams(use_tc_tiling_on_sc=False),
  )
  def gather_kernel(table_hbm, data_hbm, out_hbm, idx_v, rows, gsem, ssem):
    wid = lax.axis_index("s") * NC + lax.axis_index("c")
    row0 = wid * rows_per_w
    pltpu.sync_copy(data_hbm.at[pl.ds(row0, rows_per_w)], idx_v)

    def out_slice(g):
      return out_hbm.at[pl.ds(row0 + g * G, G), pl.ds(0, B1)]

    def fire_group(g, b):
      for k in range(G):
        pltpu.async_copy(
            table_hbm.at[idx_v.at[g * G + k]], rows[b].at[k], gsem[b])

    def drain_group(g, b):
      pltpu.make_async_copy(out_slice(g), rows[b], gsem[b]).wait()

    def store_group(g, b):
      pltpu.async_copy(rows[b], out_slice(g), ssem[b])

    def wait_store(g, b):
      pltpu.make_async_copy(rows[b], out_slice(g), ssem[b]).wait()

    for b in range(NBUF):
      fire_group(b, b)

    def step(p, carry):
      for b in range(NBUF):
        g = p * NBUF + b
        drain_group(g, b)
        store_group(g, b)

        @pl.when(g + NBUF < n_groups)
        def _():
          wait_store(g, b)
          fire_group(g + NBUF, b)

      return carry

    lax.fori_loop(0, n_groups // NBUF, step, 0)
    for b in range(NBUF):
      wait_store(n_groups - NBUF + b, b)

  return gather_kernel


def kernel(data, ivectors):
  B0, B1 = data.shape
  V, D = ivectors.shape
  tpad = jnp.pad(ivectors, ((0, 0), (0, PADD - D)))
  out = _build(V, D, B0, B1)(tpad, data.astype(jnp.int32))
  return out[:, :B1, :D]


# speedup vs baseline: 1.4457x; 1.1695x over previous
import functools

import jax
import jax.numpy as jnp
from jax import lax
from jax.experimental import pallas as pl
from jax.experimental.pallas import tpu as pltpu
from jax.experimental.pallas import tpu_sc as plsc

NC = 2
NS = 16
NW = NC * NS

G = 16
NBUF = 2
PADR = 56
PADD = 128


@functools.cache
def _build(V, D, B0, B1):
  assert B0 % NW == 0
  rows_per_w = B0 // NW
  assert rows_per_w % (G * NBUF) == 0
  n_groups = rows_per_w // G
  mesh = plsc.VectorSubcoreMesh(
      core_axis_name="c", subcore_axis_name="s", num_cores=NC, num_subcores=NS)

  @functools.partial(
      pl.kernel,
      out_type=jax.ShapeDtypeStruct((B0, PADR, PADD), jnp.float32),
      mesh=mesh,
      scratch_types=[
          pltpu.VMEM((rows_per_w, B1), jnp.int32),
          [pltpu.VMEM((G, B1, D), jnp.float32) for _ in range(NBUF)],
          [pltpu.SemaphoreType.DMA for _ in range(NBUF)],
          [pltpu.SemaphoreType.DMA for _ in range(NBUF)],
      ],
      compiler_params=pltpu.CompilerParams(use_tc_tiling_on_sc=False),
  )
  def gather_kernel(table_hbm, data_hbm, out_hbm, idx_v, rows, gsem, ssem):
    wid = lax.axis_index("s") * NC + lax.axis_index("c")
    row0 = wid * rows_per_w
    pltpu.sync_copy(data_hbm.at[pl.ds(row0, rows_per_w)], idx_v)

    def out_slice(g):
      return out_hbm.at[pl.ds(row0 + g * G, G), pl.ds(0, B1), pl.ds(0, D)]

    def fire_group(g, b):
      for k in range(G):
        pltpu.async_copy(
            table_hbm.at[idx_v.at[g * G + k]], rows[b].at[k], gsem[b])

    def drain_group(g, b):
      for k in range(G):
        pltpu.make_async_copy(
            table_hbm.at[idx_v.at[g * G + k]], rows[b].at[k], gsem[b]).wait()

    def store_group(g, b):
      pltpu.async_copy(rows[b], out_slice(g), ssem[b])

    def wait_store(g, b):
      pltpu.make_async_copy(rows[b], out_slice(g), ssem[b]).wait()

    for b in range(NBUF):
      fire_group(b, b)

    def step(p, carry):
      for b in range(NBUF):
        g = p * NBUF + b
        drain_group(g, b)
        store_group(g, b)

        @pl.when(g + NBUF < n_groups)
        def _():
          wait_store(g, b)
          fire_group(g + NBUF, b)

      return carry

    lax.fori_loop(0, n_groups // NBUF, step, 0)
    for b in range(NBUF):
      wait_store(n_groups - NBUF + b, b)

  return gather_kernel


def kernel(data, ivectors):
  B0, B1 = data.shape
  V, D = ivectors.shape
  tpad = jnp.pad(ivectors, ((0, 0), (0, PADD - D))).reshape(V * (PADD // D), D)
  data2 = data.astype(jnp.int32) * (PADD // D)
  out = _build(V, D, B0, B1)(tpad, data2)
  return out[:, :B1, :D]
